# baseline (speedup 1.0000x reference)
import functools

import jax
import jax.numpy as jnp
from jax.experimental import pallas as pl
from jax.experimental.pallas import tpu as pltpu


def _mm_kernel(x_hbm, w_ref, b_ref, o_ref, xbuf, sem):
    cp = pltpu.make_async_copy(x_hbm, xbuf, sem)
    cp.start()
    cp.wait()
    o_ref[...] = (
        jnp.dot(xbuf[...], w_ref[...], preferred_element_type=jnp.float32)
        + b_ref[...]
    )


@functools.partial(jax.jit, static_argnames=())
def kernel(x, W, b):
    B, V = x.shape
    D = W.shape[1]
    b2 = b.reshape(1, D)
    out = pl.pallas_call(
        _mm_kernel,
        in_specs=[
            pl.BlockSpec(memory_space=pltpu.MemorySpace.HBM),
            pl.BlockSpec((V, D), lambda: (0, 0)),
            pl.BlockSpec((1, D), lambda: (0, 0)),
        ],
        out_specs=pl.BlockSpec((B, D), lambda: (0, 0)),
        out_shape=jax.ShapeDtypeStruct((B, D), jnp.float32),
        scratch_shapes=[
            pltpu.VMEM((B, V), jnp.float32),
            pltpu.SemaphoreType.DMA,
        ],
    )(x, W, b2)
    return out
